# baseline (speedup 1.0000x reference)
import functools

import jax
import jax.numpy as jnp
from jax import lax
from jax.experimental import pallas as pl
from jax.experimental.pallas import tpu as pltpu
from jax.experimental.pallas import tpu_sc as plsc

B = 16384
EMB = 16
HID = 32
FEAT = 128
CHUNK = 128
GROUP = 16


def _make_gather():
    info = plsc.get_sparse_core_info()
    nw = info.num_cores * info.num_subcores
    b_per_w = B // nw
    n_chunks = b_per_w // CHUNK
    n_groups = b_per_w // GROUP
    mesh = plsc.VectorSubcoreMesh(core_axis_name="c", subcore_axis_name="s")

    @functools.partial(
        pl.kernel,
        mesh=mesh,
        out_type=jax.ShapeDtypeStruct((B, FEAT), jnp.float32),
        scratch_types=[
            pltpu.VMEM((b_per_w,), jnp.int32),
            pltpu.VMEM((b_per_w,), jnp.int32),
            pltpu.VMEM((b_per_w,), jnp.int32),
            pltpu.VMEM((b_per_w,), jnp.int32),
            pltpu.VMEM((CHUNK, FEAT), jnp.float32),
            pltpu.VMEM((CHUNK, FEAT), jnp.float32),
            pltpu.VMEM((CHUNK, FEAT), jnp.float32),
            pltpu.SemaphoreType.DMA,
        ],
        compiler_params=pltpu.CompilerParams(needs_layout_passes=False),
    )
    def gather(uidx_hbm, qidx_hbm, utab_hbm, qtab_hbm, x_hbm,
               uidx_v, qidx_v, util_v, qtil_v, utiles, qtiles, xbuf, sem):
        wid = lax.axis_index("s") * info.num_cores + lax.axis_index("c")
        base = wid * b_per_w
        pltpu.sync_copy(uidx_hbm.at[pl.ds(base, b_per_w)], uidx_v)
        pltpu.sync_copy(qidx_hbm.at[pl.ds(base, b_per_w)], qidx_v)
        for i in range(n_groups):
            s = pl.ds(i * GROUP, GROUP)
            util_v[s] = lax.shift_right_logical(uidx_v[s], 3)
            qtil_v[s] = lax.shift_right_logical(qidx_v[s], 3)

        @pl.loop(0, n_chunks)
        def chunk_body(g):
            cbase = g * CHUNK
            cu = pltpu.async_copy(
                utab_hbm.at[util_v.at[pl.ds(cbase, CHUNK)]], utiles, sem)
            cq = pltpu.async_copy(
                qtab_hbm.at[qtil_v.at[pl.ds(cbase, CHUNK)]], qtiles, sem)
            cu.wait()
            cq.wait()
            for k in range(CHUNK // GROUP):
                off = cbase + k * GROUP
                rows = lax.iota(jnp.int32, GROUP) + (k * GROUP)
                uoff = (uidx_v[pl.ds(off, GROUP)] & 7) * EMB
                qoff = (qidx_v[pl.ds(off, GROUP)] & 7) * EMB
                for f in range(EMB):
                    lane = jnp.full((GROUP,), f, jnp.int32)
                    vu = plsc.load_gather(utiles, [rows, uoff + f])
                    plsc.store_scatter(xbuf, [rows, lane], vu)
                    vq = plsc.load_gather(qtiles, [rows, qoff + f])
                    plsc.store_scatter(xbuf, [rows, lane + EMB], vq)
            pltpu.sync_copy(xbuf, x_hbm.at[pl.ds(base + cbase, CHUNK)])

    return gather


_gather = _make_gather()


def _mlp_body(x_ref, t_ref, w1_ref, w1t_ref, b1_ref, w2_ref, b2_ref, o_ref):
    lane = lax.broadcasted_iota(jnp.int32, (B, FEAT), 1)
    x = jnp.where(lane < 2 * EMB, x_ref[...], 0.0)
    h = (jnp.dot(x, w1_ref[...], preferred_element_type=jnp.float32)
         + t_ref[...] * w1t_ref[...]
         + b1_ref[...])
    h = jnp.maximum(h, 0.0)
    o = jnp.dot(h, w2_ref[...], preferred_element_type=jnp.float32) + b2_ref[...]
    o_ref[...] = jax.nn.sigmoid(o)


_mlp = pl.pallas_call(
    _mlp_body,
    out_shape=jax.ShapeDtypeStruct((B, 1), jnp.float32),
)


def kernel(user, quiz, time, user_table, quiz_table, W1, b1, W2, b2):
    x = _gather(user.astype(jnp.int32), quiz.astype(jnp.int32),
                user_table.reshape(-1, FEAT), quiz_table.reshape(-1, FEAT))
    w1p = jnp.zeros((FEAT, HID), jnp.float32).at[:2 * EMB].set(W1[:2 * EMB])
    out = _mlp(x, time, w1p, W1[2 * EMB:], b1.reshape(1, HID), W2,
               b2.reshape(1, 1))
    return out.reshape(B)


# speedup vs baseline: 1.5428x; 1.5428x over previous
import functools

import jax
import jax.numpy as jnp
from jax import lax
from jax.experimental import pallas as pl
from jax.experimental.pallas import tpu as pltpu
from jax.experimental.pallas import tpu_sc as plsc

B = 16384
EMB = 16
HID = 32
CHUNK = 64
GROUP = 16


def _make_gather():
    info = plsc.get_sparse_core_info()
    nw = info.num_cores * info.num_subcores
    b_per_w = B // nw
    n_chunks = b_per_w // CHUNK
    mesh = plsc.VectorSubcoreMesh(core_axis_name="c", subcore_axis_name="s")

    @functools.partial(
        pl.kernel,
        mesh=mesh,
        out_type=[
            jax.ShapeDtypeStruct((B, EMB), jnp.float32),
            jax.ShapeDtypeStruct((B, EMB), jnp.float32),
        ],
        scratch_types=[
            pltpu.VMEM((b_per_w,), jnp.int32),
            pltpu.VMEM((b_per_w,), jnp.int32),
            pltpu.VMEM((CHUNK, EMB), jnp.float32),
            pltpu.VMEM((CHUNK, EMB), jnp.float32),
            pltpu.SemaphoreType.DMA,
        ],
        compiler_params=pltpu.CompilerParams(needs_layout_passes=False),
    )
    def gather(uidx_hbm, qidx_hbm, utab_hbm, qtab_hbm, u_hbm, q_hbm,
               uidx_v, qidx_v, ubuf, qbuf, sem):
        wid = lax.axis_index("s") * info.num_cores + lax.axis_index("c")
        base = wid * b_per_w
        pltpu.sync_copy(uidx_hbm.at[pl.ds(base, b_per_w)], uidx_v)
        pltpu.sync_copy(qidx_hbm.at[pl.ds(base, b_per_w)], qidx_v)

        @pl.loop(0, n_chunks)
        def chunk_body(g):
            cbase = g * CHUNK
            copies = []
            for k in range(CHUNK // GROUP):
                off = cbase + k * GROUP
                uvec = uidx_v[pl.ds(off, GROUP)]
                qvec = qidx_v[pl.ds(off, GROUP)]
                for l in range(GROUP):
                    row = k * GROUP + l
                    copies.append(pltpu.async_copy(
                        utab_hbm.at[pl.ds(uvec[l], 1)],
                        ubuf.at[pl.ds(row, 1)], sem))
                    copies.append(pltpu.async_copy(
                        qtab_hbm.at[pl.ds(qvec[l], 1)],
                        qbuf.at[pl.ds(row, 1)], sem))
            for c in copies:
                c.wait()
            pltpu.sync_copy(ubuf, u_hbm.at[pl.ds(base + cbase, CHUNK)])
            pltpu.sync_copy(qbuf, q_hbm.at[pl.ds(base + cbase, CHUNK)])

    return gather


_gather = _make_gather()


def _mlp_body(u_ref, q_ref, t_ref, w1u_ref, w1q_ref, w1t_ref, b1_ref,
              w2_ref, b2_ref, o_ref):
    h = (jnp.dot(u_ref[...], w1u_ref[...], preferred_element_type=jnp.float32)
         + jnp.dot(q_ref[...], w1q_ref[...], preferred_element_type=jnp.float32)
         + t_ref[...] * w1t_ref[...]
         + b1_ref[...])
    h = jnp.maximum(h, 0.0)
    o = jnp.dot(h, w2_ref[...], preferred_element_type=jnp.float32) + b2_ref[...]
    o_ref[...] = jax.nn.sigmoid(o)


_mlp = pl.pallas_call(
    _mlp_body,
    out_shape=jax.ShapeDtypeStruct((B, 1), jnp.float32),
)


def kernel(user, quiz, time, user_table, quiz_table, W1, b1, W2, b2):
    u, q = _gather(user.astype(jnp.int32), quiz.astype(jnp.int32),
                   user_table, quiz_table)
    out = _mlp(u, q, time,
               W1[:EMB], W1[EMB:2 * EMB], W1[2 * EMB:],
               b1.reshape(1, HID), W2, b2.reshape(1, 1))
    return out.reshape(B)


# speedup vs baseline: 1.5455x; 1.0017x over previous
import functools

import jax
import jax.numpy as jnp
from jax import lax
from jax.experimental import pallas as pl
from jax.experimental.pallas import tpu as pltpu
from jax.experimental.pallas import tpu_sc as plsc

B = 16384
EMB = 16
HID = 32
CHUNK = 64
GROUP = 16


def _make_gather():
    info = plsc.get_sparse_core_info()
    nw = info.num_cores * info.num_subcores
    b_per_w = B // nw
    n_chunks = b_per_w // CHUNK
    mesh = plsc.VectorSubcoreMesh(core_axis_name="c", subcore_axis_name="s")

    @functools.partial(
        pl.kernel,
        mesh=mesh,
        out_type=[
            jax.ShapeDtypeStruct((B, EMB), jnp.float32),
            jax.ShapeDtypeStruct((B, EMB), jnp.float32),
        ],
        scratch_types=[
            pltpu.VMEM((b_per_w,), jnp.int32),
            pltpu.VMEM((b_per_w,), jnp.int32),
            pltpu.VMEM((CHUNK, EMB), jnp.float32),
            pltpu.VMEM((CHUNK, EMB), jnp.float32),
            pltpu.SemaphoreType.DMA,
        ],
        compiler_params=pltpu.CompilerParams(
            needs_layout_passes=False, use_tc_tiling_on_sc=True),
    )
    def gather(uidx_hbm, qidx_hbm, utab_hbm, qtab_hbm, u_hbm, q_hbm,
               uidx_v, qidx_v, ubuf, qbuf, sem):
        wid = lax.axis_index("s") * info.num_cores + lax.axis_index("c")
        base = wid * b_per_w
        pltpu.sync_copy(uidx_hbm.at[pl.ds(base, b_per_w)], uidx_v)
        pltpu.sync_copy(qidx_hbm.at[pl.ds(base, b_per_w)], qidx_v)

        @pl.loop(0, n_chunks)
        def chunk_body(g):
            cbase = g * CHUNK
            copies = []
            for k in range(CHUNK // GROUP):
                off = cbase + k * GROUP
                uvec = uidx_v[pl.ds(off, GROUP)]
                qvec = qidx_v[pl.ds(off, GROUP)]
                for l in range(GROUP):
                    row = k * GROUP + l
                    copies.append(pltpu.async_copy(
                        utab_hbm.at[pl.ds(uvec[l], 1)],
                        ubuf.at[pl.ds(row, 1)], sem))
                    copies.append(pltpu.async_copy(
                        qtab_hbm.at[pl.ds(qvec[l], 1)],
                        qbuf.at[pl.ds(row, 1)], sem))
            for c in copies:
                c.wait()
            pltpu.sync_copy(ubuf, u_hbm.at[pl.ds(base + cbase, CHUNK)])
            pltpu.sync_copy(qbuf, q_hbm.at[pl.ds(base + cbase, CHUNK)])

    return gather


_gather = _make_gather()


def _mlp_body(u_ref, q_ref, t_ref, w1u_ref, w1q_ref, w1t_ref, b1_ref,
              w2_ref, b2_ref, o_ref):
    h = (jnp.dot(u_ref[...], w1u_ref[...], preferred_element_type=jnp.float32)
         + jnp.dot(q_ref[...], w1q_ref[...], preferred_element_type=jnp.float32)
         + t_ref[...] * w1t_ref[...]
         + b1_ref[...])
    h = jnp.maximum(h, 0.0)
    o = jnp.dot(h, w2_ref[...], preferred_element_type=jnp.float32) + b2_ref[...]
    o_ref[...] = jax.nn.sigmoid(o)


_mlp = pl.pallas_call(
    _mlp_body,
    out_shape=jax.ShapeDtypeStruct((B, 1), jnp.float32),
)


def kernel(user, quiz, time, user_table, quiz_table, W1, b1, W2, b2):
    u, q = _gather(user.astype(jnp.int32), quiz.astype(jnp.int32),
                   user_table, quiz_table)
    out = _mlp(u, q, time,
               W1[:EMB], W1[EMB:2 * EMB], W1[2 * EMB:],
               b1.reshape(1, HID), W2, b2.reshape(1, 1))
    return out.reshape(B)
